# baseline (speedup 1.0000x reference)
import functools

import jax
import jax.numpy as jnp
from jax import lax
from jax.experimental import pallas as pl
from jax.experimental.pallas import tpu as pltpu
from jax.experimental.pallas import tpu_sc as plsc

N = 10000
E = 160000
NODE_DIM = 256
EDGE_DIM = 16
HID = 512
TDIM = 128
G = 32

NC = 2
NS = 16
NP = N + 16
ISTRIPE = NP // NS
OSTRIPE = N // NS
EP = 163840
EPT = EP // NS
C1 = 80
GB1 = 8
NG1 = EPT // C1 // GB1
C2 = 128
GB2 = 4
NG2 = EPT // C2 // GB2
CH = 128
SCH = 40
UD = 288
UH = UD // 2
ZH = NODE_DIM // 2

_mesh = lambda: plsc.VectorSubcoreMesh(core_axis_name="c", subcore_axis_name="s")
_sc_params = lambda: pltpu.CompilerParams(use_tc_tiling_on_sc=False)


def _edge_pipeline(src_hbm, idx_hbm, acc, idxg, rows, gsems, ssems,
                   ibase, ngrp, gb):
    nb = len(rows)

    def body(g, carry):
        pltpu.sync_copy(idx_hbm.at[ibase + g], idxg)
        gd = [None] * gb
        sd = [None] * gb
        for j in range(min(nb, gb)):
            gd[j] = pltpu.async_copy(src_hbm.at[idxg.at[2 * j]],
                                     rows[j % nb], gsems[j % nb])
        for b in range(gb):
            if b >= nb - 1:
                sd[b - nb + 1].wait()
                jj = b + 1
                if jj < gb:
                    gd[jj] = pltpu.async_copy(src_hbm.at[idxg.at[2 * jj]],
                                              rows[jj % nb], gsems[jj % nb])
            gd[b].wait()
            sd[b] = pltpu.async_copy(rows[b % nb], acc.at[idxg.at[2 * b + 1]],
                                     ssems[b % nb], add=True)
        for j in range(max(0, gb - nb + 1), gb):
            sd[j].wait()
        return carry

    lax.fori_loop(0, ngrp, body, 0)


def _sc_pass1(u2, idx1, za):
    @functools.partial(
        pl.kernel,
        out_type=(
            jax.ShapeDtypeStruct((N, UH), jnp.float32),
            jax.ShapeDtypeStruct((N, UH), jnp.float32),
        ),
        mesh=_mesh(),
        compiler_params=_sc_params(),
        scratch_types=[
            pltpu.VMEM((2 * GB1, C1), jnp.int32),
            pltpu.VMEM((C1, UH), jnp.float32),
            pltpu.VMEM((C1, UH), jnp.float32),
            pltpu.VMEM((C1, UH), jnp.float32),
            pltpu.VMEM_SHARED((NP, UH), jnp.float32),
            pltpu.SemaphoreType.DMA,
            pltpu.SemaphoreType.DMA,
            pltpu.SemaphoreType.DMA,
            pltpu.SemaphoreType.DMA,
            pltpu.SemaphoreType.DMA,
            pltpu.SemaphoreType.DMA,
        ],
    )
    def k(u2_hbm, idx1_hbm, za_hbm, axc0_out, axc1_out,
          idxg, rows_a, rows_b, rows_c, acc,
          gsem_a, gsem_b, gsem_c, ssem_a, ssem_b, ssem_c):
        c = lax.axis_index("c")
        s = lax.axis_index("s")
        pltpu.sync_copy(za_hbm, acc.at[pl.ds(s * ISTRIPE, ISTRIPE)])
        plsc.subcore_barrier()
        _edge_pipeline(u2_hbm, idx1_hbm, acc, idxg,
                       (rows_a, rows_b, rows_c), (gsem_a, gsem_b, gsem_c),
                       (ssem_a, ssem_b, ssem_c), (c * NS + s) * NG1, NG1, GB1)
        plsc.subcore_barrier()

        @pl.when(c == 0)
        def _():
            pltpu.sync_copy(acc.at[pl.ds(s * OSTRIPE, OSTRIPE)],
                            axc0_out.at[pl.ds(s * OSTRIPE, OSTRIPE)])

        @pl.when(c == 1)
        def _():
            pltpu.sync_copy(acc.at[pl.ds(s * OSTRIPE, OSTRIPE)],
                            axc1_out.at[pl.ds(s * OSTRIPE, OSTRIPE)])

    return k(u2, idx1, za)


def _sc_spass(ea, dst2d, zs):
    @functools.partial(
        pl.kernel,
        out_type=(
            jax.ShapeDtypeStruct((N, EDGE_DIM), jnp.float32),
            jax.ShapeDtypeStruct((N, EDGE_DIM), jnp.float32),
        ),
        mesh=_mesh(),
        compiler_params=_sc_params(),
        scratch_types=[
            pltpu.VMEM((SCH, CH), jnp.int32),
            pltpu.VMEM((CH, EDGE_DIM), jnp.float32),
            pltpu.VMEM((CH, EDGE_DIM), jnp.float32),
            pltpu.VMEM_SHARED((NP, EDGE_DIM), jnp.float32),
            pltpu.SemaphoreType.DMA,
            pltpu.SemaphoreType.DMA,
            pltpu.SemaphoreType.DMA,
            pltpu.SemaphoreType.DMA,
        ],
    )
    def k(ea_hbm, dst_hbm, zs_hbm, s0_out, s1_out,
          dbuf, ea_a, ea_b, acc_s, gsem_a, gsem_b, ssem_a, ssem_b):
        c = lax.axis_index("c")
        s = lax.axis_index("s")
        pltpu.sync_copy(zs_hbm, acc_s.at[pl.ds(s * ISTRIPE, ISTRIPE)])
        pltpu.sync_copy(dst_hbm.at[pl.ds((c * NS + s) * SCH, SCH)], dbuf)
        plsc.subcore_barrier()

        ebase = (c * NS + s) * (SCH * CH)

        def body(i, carry):
            j0 = 2 * i
            g_a = pltpu.async_copy(ea_hbm.at[pl.ds(ebase + j0 * CH, CH)],
                                   ea_a, gsem_a)
            g_b = pltpu.async_copy(ea_hbm.at[pl.ds(ebase + (j0 + 1) * CH, CH)],
                                   ea_b, gsem_b)
            g_a.wait()
            s_a = pltpu.async_copy(ea_a, acc_s.at[dbuf.at[j0]], ssem_a, add=True)
            g_b.wait()
            s_b = pltpu.async_copy(ea_b, acc_s.at[dbuf.at[j0 + 1]], ssem_b, add=True)
            s_a.wait()
            s_b.wait()
            return carry

        lax.fori_loop(0, SCH // 2, body, 0)
        plsc.subcore_barrier()

        @pl.when(c == 0)
        def _():
            pltpu.sync_copy(acc_s.at[pl.ds(s * OSTRIPE, OSTRIPE)],
                            s0_out.at[pl.ds(s * OSTRIPE, OSTRIPE)])

        @pl.when(c == 1)
        def _():
            pltpu.sync_copy(acc_s.at[pl.ds(s * OSTRIPE, OSTRIPE)],
                            s1_out.at[pl.ds(s * OSTRIPE, OSTRIPE)])

    return k(ea, dst2d, zs)


def _sc_pass2(z2, idx2, b0, b1):
    @functools.partial(
        pl.kernel,
        out_type=(
            jax.ShapeDtypeStruct((N, ZH), jnp.float32),
            jax.ShapeDtypeStruct((N, ZH), jnp.float32),
        ),
        mesh=_mesh(),
        compiler_params=_sc_params(),
        scratch_types=[
            pltpu.VMEM((2 * GB2, C2), jnp.int32),
            pltpu.VMEM((C2, ZH), jnp.float32),
            pltpu.VMEM((C2, ZH), jnp.float32),
            pltpu.VMEM((C2, ZH), jnp.float32),
            pltpu.VMEM_SHARED((NP, ZH), jnp.float32),
            pltpu.SemaphoreType.DMA,
            pltpu.SemaphoreType.DMA,
            pltpu.SemaphoreType.DMA,
            pltpu.SemaphoreType.DMA,
            pltpu.SemaphoreType.DMA,
            pltpu.SemaphoreType.DMA,
        ],
    )
    def k(z2_hbm, idx2_hbm, b0_hbm, b1_hbm, out0_hbm, out1_hbm,
          idxg, rows_a, rows_b, rows_c, acc,
          gsem_a, gsem_b, gsem_c, ssem_a, ssem_b, ssem_c):
        c = lax.axis_index("c")
        s = lax.axis_index("s")
        @pl.when(c == 0)
        def _():
            pltpu.sync_copy(b0_hbm.at[pl.ds(s * OSTRIPE, OSTRIPE)],
                            acc.at[pl.ds(s * OSTRIPE, OSTRIPE)])

        @pl.when(c == 1)
        def _():
            pltpu.sync_copy(b1_hbm.at[pl.ds(s * OSTRIPE, OSTRIPE)],
                            acc.at[pl.ds(s * OSTRIPE, OSTRIPE)])

        plsc.subcore_barrier()
        _edge_pipeline(z2_hbm, idx2_hbm, acc, idxg,
                       (rows_a, rows_b, rows_c), (gsem_a, gsem_b, gsem_c),
                       (ssem_a, ssem_b, ssem_c), (c * NS + s) * NG2, NG2, GB2)
        plsc.subcore_barrier()

        @pl.when(c == 0)
        def _():
            pltpu.sync_copy(acc.at[pl.ds(s * OSTRIPE, OSTRIPE)],
                            out0_hbm.at[pl.ds(s * OSTRIPE, OSTRIPE)])

        @pl.when(c == 1)
        def _():
            pltpu.sync_copy(acc.at[pl.ds(s * OSTRIPE, OSTRIPE)],
                            out1_hbm.at[pl.ds(s * OSTRIPE, OSTRIPE)])

    return k(z2, idx2, b0, b1)


def _prep_weights(t2d, n1_W, e1_W, e1_b2, n2_W, n2_b2, out_W, out_b2, e2_W, e2_b2):
    def body(t_ref, n1_ref, e1w_ref, e1b_ref, n2_ref, n2b_ref, ow_ref, ob_ref,
             e2w_ref, e2b_ref, t2_ref, e1_ref, b1e_ref, m_ref, k2_ref, c2v_ref,
             row2_ref):
        half = TDIM // 2
        k = lax.broadcasted_iota(jnp.int32, (1, half), 1).astype(jnp.float32)
        freq = jnp.exp(-k * (10000.0 ** (-2.0 / half)))
        emb = t_ref[...] * freq
        T = jnp.concatenate([jnp.sin(emb), jnp.cos(emb)], axis=1)
        n1 = n1_ref[...]
        w1b = n1[NODE_DIM:NODE_DIM + TDIM, :]
        n1bot = n1[NODE_DIM + TDIM:, :]
        t2_ref[...] = jnp.dot(T, w1b, preferred_element_type=jnp.float32)
        e1_ref[...] = jnp.dot(e1w_ref[...], n1bot, preferred_element_type=jnp.float32)
        b1e_ref[...] = jnp.dot(e1b_ref[...], n1bot, preferred_element_type=jnp.float32)
        n2 = n2_ref[...]
        ow = ow_ref[...]
        m_ref[...] = jnp.dot(n2[:HID, :], ow, preferred_element_type=jnp.float32)
        wb2o = jnp.dot(n2[HID:, :], ow, preferred_element_type=jnp.float32)
        k2_ref[...] = jnp.dot(e2w_ref[...], wb2o, preferred_element_type=jnp.float32)
        c2v_ref[...] = jnp.dot(e2b_ref[...], wb2o, preferred_element_type=jnp.float32)
        row2_ref[...] = jnp.dot(n2b_ref[...], ow, preferred_element_type=jnp.float32) + ob_ref[...]

    outs = pl.pallas_call(
        body,
        out_shape=(
            jax.ShapeDtypeStruct((G, HID), jnp.float32),
            jax.ShapeDtypeStruct((EDGE_DIM, HID), jnp.float32),
            jax.ShapeDtypeStruct((1, HID), jnp.float32),
            jax.ShapeDtypeStruct((HID, NODE_DIM), jnp.float32),
            jax.ShapeDtypeStruct((EDGE_DIM, NODE_DIM), jnp.float32),
            jax.ShapeDtypeStruct((1, NODE_DIM), jnp.float32),
            jax.ShapeDtypeStruct((1, NODE_DIM), jnp.float32),
        ),
    )(t2d, n1_W, e1_W, e1_b2, n2_W, n2_b2, out_W, out_b2, e2_W, e2_b2)
    return outs


def _dense_mid(a0, a1, s0, s1, w1aa, wcomb, e1, b1e, n1_b2, m, k2, c2v, row2):
    BN = 400
    nblocks = N // BN

    def body(a0_ref, a1_ref, s0_ref, s1_ref, w1aa_ref, wcomb_ref, e1_ref,
             b1e_ref, n1b_ref, m_ref, k2_ref, c2v_ref, row2_ref,
             z_ref, b0_ref, b1_ref):
        a0b = a0_ref[...]
        a1b = a1_ref[...]
        sv = s0_ref[...] + s1_ref[...]
        deg = jnp.sum(a1b[:, UH - G:], axis=1, keepdims=True)
        h1 = (jnp.dot(a0b, w1aa_ref[...], preferred_element_type=jnp.float32)
              + jnp.dot(a1b, wcomb_ref[...], preferred_element_type=jnp.float32)
              + jnp.dot(sv, e1_ref[...], preferred_element_type=jnp.float32)
              + deg * b1e_ref[...]
              + n1b_ref[...])
        z_ref[...] = jnp.dot(h1, m_ref[...], preferred_element_type=jnp.float32)
        base = (jnp.dot(sv, k2_ref[...], preferred_element_type=jnp.float32)
                + deg * c2v_ref[...] + row2_ref[...])
        b0_ref[...] = base[:, :ZH]
        b1_ref[...] = base[:, ZH:]

    full = lambda shape: pl.BlockSpec(shape, lambda i: (0, 0))
    z, b0, b1 = pl.pallas_call(
        body,
        grid=(nblocks,),
        in_specs=[
            pl.BlockSpec((BN, UH), lambda i: (i, 0)),
            pl.BlockSpec((BN, UH), lambda i: (i, 0)),
            pl.BlockSpec((BN, EDGE_DIM), lambda i: (i, 0)),
            pl.BlockSpec((BN, EDGE_DIM), lambda i: (i, 0)),
            full((UH, HID)),
            full((UH, HID)),
            full((EDGE_DIM, HID)),
            full((1, HID)),
            full((1, HID)),
            full((HID, NODE_DIM)),
            full((EDGE_DIM, NODE_DIM)),
            full((1, NODE_DIM)),
            full((1, NODE_DIM)),
        ],
        out_specs=(
            pl.BlockSpec((BN, NODE_DIM), lambda i: (i, 0)),
            pl.BlockSpec((BN, ZH), lambda i: (i, 0)),
            pl.BlockSpec((BN, ZH), lambda i: (i, 0)),
        ),
        out_shape=(
            jax.ShapeDtypeStruct((N, NODE_DIM), jnp.float32),
            jax.ShapeDtypeStruct((N, ZH), jnp.float32),
            jax.ShapeDtypeStruct((N, ZH), jnp.float32),
        ),
    )(a0, a1, s0, s1, w1aa, wcomb, e1, b1e, n1_b2, m, k2, c2v, row2)
    return z, b0, b1


def kernel(x, edge_index, edge_attr, t, batch,
           e1_W, e1_b, n1_W, n1_b,
           e2_W, e2_b, n2_W, n2_b,
           out_W, out_b):
    f32 = jnp.float32
    src = edge_index[0].astype(jnp.int32)
    dst = edge_index[1].astype(jnp.int32)
    batch = batch.astype(jnp.int32)

    onehot = (batch[:, None] == jnp.arange(G, dtype=jnp.int32)[None, :]).astype(f32)
    u2 = jnp.concatenate([x, onehot], axis=1).reshape(2 * N, UH)
    pad_e = EP - E
    src_p = jnp.concatenate([src, jnp.zeros((pad_e,), jnp.int32)])
    trash = N + (jnp.arange(pad_e, dtype=jnp.int32) % 16)
    dst_p = jnp.concatenate([dst, trash])
    ea_p = jnp.concatenate([edge_attr, jnp.zeros((pad_e, EDGE_DIM), f32)])

    def grouped_idx(ngrp, gb, ch):
        st = src_p.reshape(NS, ngrp, gb, ch)
        dt = dst_p.reshape(NS, ngrp, gb, ch)
        per_c = [jnp.stack([2 * st + c, dt], axis=3) for c in range(NC)]
        return jnp.concatenate(per_c).reshape(NC * NS * ngrp, 2 * gb, ch)

    idx1 = grouped_idx(NG1, GB1, C1)
    idx2 = grouped_idx(NG2, GB2, C2)
    dst2d = dst_p.reshape(NC * NS * SCH, CH)

    za = jnp.zeros((ISTRIPE, UH), f32)
    zs = jnp.zeros((ISTRIPE, EDGE_DIM), f32)

    a0, a1 = _sc_pass1(u2, idx1, za)
    s0, s1 = _sc_spass(ea_p, dst2d, zs)

    t2, e1p, b1e, m, k2, c2v, row2 = _prep_weights(
        t.reshape(G, 1), n1_W, e1_W, e1_b.reshape(1, -1), n2_W,
        n2_b.reshape(1, -1), out_W, out_b.reshape(1, -1), e2_W,
        e2_b.reshape(1, -1))

    w1aa = n1_W[:UH, :]
    wcomb = jnp.concatenate([n1_W[UH:NODE_DIM, :], t2], axis=0)

    z, b0, b1 = _dense_mid(a0, a1, s0, s1, w1aa, wcomb, e1p, b1e,
                           n1_b.reshape(1, -1), m, k2, c2v, row2)

    z2 = z.reshape(2 * N, ZH)
    o0, o1 = _sc_pass2(z2, idx2, b0, b1)
    return jnp.concatenate([o0, o1], axis=1)


# speedup vs baseline: 1.1394x; 1.1394x over previous
import functools

import jax
import jax.numpy as jnp
from jax import lax
from jax.experimental import pallas as pl
from jax.experimental.pallas import tpu as pltpu
from jax.experimental.pallas import tpu_sc as plsc

N = 10000
E = 160000
NODE_DIM = 256
EDGE_DIM = 16
HID = 512
TDIM = 128
G = 32

NC = 2
NS = 16
NP = N + 16
ISTRIPE = NP // NS
OSTRIPE = N // NS
EP = 163840
EPT = EP // NS
CH = 128
NCH = EPT // CH
GB = 8
NGRP = NCH // GB
SCH = 40
UD = 288
UH = UD // 2
ZH = NODE_DIM // 2

_mesh = lambda: plsc.VectorSubcoreMesh(core_axis_name="c", subcore_axis_name="s")
_sc_params = lambda: pltpu.CompilerParams(use_tc_tiling_on_sc=False)


def _edge_pipeline(src_hbm, idx_hbm, acc, idxg, rows_a, rows_b,
                   gsem_a, gsem_b, ssem_a, ssem_b, ibase):
    rows = (rows_a, rows_b)
    gsems = (gsem_a, gsem_b)
    ssems = (ssem_a, ssem_b)

    def body(g, carry):
        pltpu.sync_copy(idx_hbm.at[ibase + g], idxg)
        gd = [None] * GB
        sd = [None] * GB
        gd[0] = pltpu.async_copy(src_hbm.at[idxg.at[0]], rows_a, gsem_a)
        gd[1] = pltpu.async_copy(src_hbm.at[idxg.at[2]], rows_b, gsem_b)
        for b in range(GB):
            if b >= 1:
                sd[b - 1].wait()
                if b + 1 < GB:
                    gd[b + 1] = pltpu.async_copy(
                        src_hbm.at[idxg.at[2 * (b + 1)]],
                        rows[(b + 1) % 2], gsems[(b + 1) % 2])
            gd[b].wait()
            sd[b] = pltpu.async_copy(rows[b % 2], acc.at[idxg.at[2 * b + 1]],
                                     ssems[b % 2], add=True)
        sd[GB - 1].wait()
        return carry

    lax.fori_loop(0, NGRP, body, 0)


def _sc_pass1(u2, idx1, za):
    @functools.partial(
        pl.kernel,
        out_type=(
            jax.ShapeDtypeStruct((N, UH), jnp.float32),
            jax.ShapeDtypeStruct((N, UH), jnp.float32),
        ),
        mesh=_mesh(),
        compiler_params=_sc_params(),
        scratch_types=[
            pltpu.VMEM((2 * GB, CH), jnp.int32),
            pltpu.VMEM((CH, UH), jnp.float32),
            pltpu.VMEM((CH, UH), jnp.float32),
            pltpu.VMEM_SHARED((NP, UH), jnp.float32),
            pltpu.SemaphoreType.DMA,
            pltpu.SemaphoreType.DMA,
            pltpu.SemaphoreType.DMA,
            pltpu.SemaphoreType.DMA,
        ],
    )
    def k(u2_hbm, idx1_hbm, za_hbm, axc0_out, axc1_out,
          idxg, rows_a, rows_b, acc, gsem_a, gsem_b, ssem_a, ssem_b):
        c = lax.axis_index("c")
        s = lax.axis_index("s")
        pltpu.sync_copy(za_hbm, acc.at[pl.ds(s * ISTRIPE, ISTRIPE)])
        plsc.subcore_barrier()
        _edge_pipeline(u2_hbm, idx1_hbm, acc, idxg, rows_a, rows_b,
                       gsem_a, gsem_b, ssem_a, ssem_b, (c * NS + s) * NGRP)
        plsc.subcore_barrier()

        @pl.when(c == 0)
        def _():
            pltpu.sync_copy(acc.at[pl.ds(s * OSTRIPE, OSTRIPE)],
                            axc0_out.at[pl.ds(s * OSTRIPE, OSTRIPE)])

        @pl.when(c == 1)
        def _():
            pltpu.sync_copy(acc.at[pl.ds(s * OSTRIPE, OSTRIPE)],
                            axc1_out.at[pl.ds(s * OSTRIPE, OSTRIPE)])

    return k(u2, idx1, za)


def _sc_spass(ea, dst2d, zs):
    @functools.partial(
        pl.kernel,
        out_type=(
            jax.ShapeDtypeStruct((N, EDGE_DIM), jnp.float32),
            jax.ShapeDtypeStruct((N, EDGE_DIM), jnp.float32),
        ),
        mesh=_mesh(),
        compiler_params=_sc_params(),
        scratch_types=[
            pltpu.VMEM((SCH, CH), jnp.int32),
            pltpu.VMEM((CH, EDGE_DIM), jnp.float32),
            pltpu.VMEM((CH, EDGE_DIM), jnp.float32),
            pltpu.VMEM_SHARED((NP, EDGE_DIM), jnp.float32),
            pltpu.SemaphoreType.DMA,
            pltpu.SemaphoreType.DMA,
            pltpu.SemaphoreType.DMA,
            pltpu.SemaphoreType.DMA,
        ],
    )
    def k(ea_hbm, dst_hbm, zs_hbm, s0_out, s1_out,
          dbuf, ea_a, ea_b, acc_s, gsem_a, gsem_b, ssem_a, ssem_b):
        c = lax.axis_index("c")
        s = lax.axis_index("s")
        pltpu.sync_copy(zs_hbm, acc_s.at[pl.ds(s * ISTRIPE, ISTRIPE)])
        pltpu.sync_copy(dst_hbm.at[pl.ds((c * NS + s) * SCH, SCH)], dbuf)
        plsc.subcore_barrier()

        ebase = (c * NS + s) * (SCH * CH)
        npair = jnp.maximum(0, jnp.minimum(SCH, (E - ebase) // CH)) // 2

        def body(i, carry):
            j0 = 2 * i
            g_a = pltpu.async_copy(ea_hbm.at[pl.ds(ebase + j0 * CH, CH)],
                                   ea_a, gsem_a)
            g_b = pltpu.async_copy(ea_hbm.at[pl.ds(ebase + (j0 + 1) * CH, CH)],
                                   ea_b, gsem_b)
            g_a.wait()
            s_a = pltpu.async_copy(ea_a, acc_s.at[dbuf.at[j0]], ssem_a, add=True)
            g_b.wait()
            s_b = pltpu.async_copy(ea_b, acc_s.at[dbuf.at[j0 + 1]], ssem_b, add=True)
            s_a.wait()
            s_b.wait()
            return carry

        lax.fori_loop(0, npair, body, 0)
        plsc.subcore_barrier()

        @pl.when(c == 0)
        def _():
            pltpu.sync_copy(acc_s.at[pl.ds(s * OSTRIPE, OSTRIPE)],
                            s0_out.at[pl.ds(s * OSTRIPE, OSTRIPE)])

        @pl.when(c == 1)
        def _():
            pltpu.sync_copy(acc_s.at[pl.ds(s * OSTRIPE, OSTRIPE)],
                            s1_out.at[pl.ds(s * OSTRIPE, OSTRIPE)])

    return k(ea, dst2d, zs)


def _sc_pass2(z2, idx2, b0, b1):
    @functools.partial(
        pl.kernel,
        out_type=jax.ShapeDtypeStruct((N, NODE_DIM), jnp.float32),
        mesh=_mesh(),
        compiler_params=_sc_params(),
        scratch_types=[
            pltpu.VMEM((2 * GB, CH), jnp.int32),
            pltpu.VMEM((CH, ZH), jnp.float32),
            pltpu.VMEM((CH, ZH), jnp.float32),
            pltpu.VMEM_SHARED((NP, ZH), jnp.float32),
            pltpu.SemaphoreType.DMA,
            pltpu.SemaphoreType.DMA,
            pltpu.SemaphoreType.DMA,
            pltpu.SemaphoreType.DMA,
        ],
    )
    def k(z2_hbm, idx2_hbm, b0_hbm, b1_hbm, out_hbm,
          idxg, rows_a, rows_b, acc, gsem_a, gsem_b, ssem_a, ssem_b):
        c = lax.axis_index("c")
        s = lax.axis_index("s")
        @pl.when(c == 0)
        def _():
            pltpu.sync_copy(b0_hbm.at[pl.ds(s * OSTRIPE, OSTRIPE)],
                            acc.at[pl.ds(s * OSTRIPE, OSTRIPE)])

        @pl.when(c == 1)
        def _():
            pltpu.sync_copy(b1_hbm.at[pl.ds(s * OSTRIPE, OSTRIPE)],
                            acc.at[pl.ds(s * OSTRIPE, OSTRIPE)])

        plsc.subcore_barrier()
        _edge_pipeline(z2_hbm, idx2_hbm, acc, idxg, rows_a, rows_b,
                       gsem_a, gsem_b, ssem_a, ssem_b, (c * NS + s) * NGRP)
        plsc.subcore_barrier()

        pltpu.sync_copy(acc.at[pl.ds(s * OSTRIPE, OSTRIPE)],
                        out_hbm.at[pl.ds(s * OSTRIPE, OSTRIPE),
                                   pl.ds(c * ZH, ZH)])

    return k(z2, idx2, b0, b1)


def _prep_weights(t2d, n1_W, e1_W, e1_b2, n2_W, n2_b2, out_W, out_b2, e2_W, e2_b2):
    def body(t_ref, n1_ref, e1w_ref, e1b_ref, n2_ref, n2b_ref, ow_ref, ob_ref,
             e2w_ref, e2b_ref, t2_ref, e1_ref, b1e_ref, m_ref, k2_ref, c2v_ref,
             row2_ref):
        half = TDIM // 2
        k = lax.broadcasted_iota(jnp.int32, (1, half), 1).astype(jnp.float32)
        freq = jnp.exp(-k * (10000.0 ** (-2.0 / half)))
        emb = t_ref[...] * freq
        T = jnp.concatenate([jnp.sin(emb), jnp.cos(emb)], axis=1)
        n1 = n1_ref[...]
        w1b = n1[NODE_DIM:NODE_DIM + TDIM, :]
        n1bot = n1[NODE_DIM + TDIM:, :]
        t2_ref[...] = jnp.dot(T, w1b, preferred_element_type=jnp.float32)
        e1_ref[...] = jnp.dot(e1w_ref[...], n1bot, preferred_element_type=jnp.float32)
        b1e_ref[...] = jnp.dot(e1b_ref[...], n1bot, preferred_element_type=jnp.float32)
        n2 = n2_ref[...]
        ow = ow_ref[...]
        m_ref[...] = jnp.dot(n2[:HID, :], ow, preferred_element_type=jnp.float32)
        wb2o = jnp.dot(n2[HID:, :], ow, preferred_element_type=jnp.float32)
        k2_ref[...] = jnp.dot(e2w_ref[...], wb2o, preferred_element_type=jnp.float32)
        c2v_ref[...] = jnp.dot(e2b_ref[...], wb2o, preferred_element_type=jnp.float32)
        row2_ref[...] = jnp.dot(n2b_ref[...], ow, preferred_element_type=jnp.float32) + ob_ref[...]

    outs = pl.pallas_call(
        body,
        out_shape=(
            jax.ShapeDtypeStruct((G, HID), jnp.float32),
            jax.ShapeDtypeStruct((EDGE_DIM, HID), jnp.float32),
            jax.ShapeDtypeStruct((1, HID), jnp.float32),
            jax.ShapeDtypeStruct((HID, NODE_DIM), jnp.float32),
            jax.ShapeDtypeStruct((EDGE_DIM, NODE_DIM), jnp.float32),
            jax.ShapeDtypeStruct((1, NODE_DIM), jnp.float32),
            jax.ShapeDtypeStruct((1, NODE_DIM), jnp.float32),
        ),
    )(t2d, n1_W, e1_W, e1_b2, n2_W, n2_b2, out_W, out_b2, e2_W, e2_b2)
    return outs


def _dense_mid(a0, a1, s0, s1, w1aa, wcomb, e1, b1e, n1_b2, m, k2, c2v, row2):
    BN = 400
    nblocks = N // BN

    def body(a0_ref, a1_ref, s0_ref, s1_ref, w1aa_ref, wcomb_ref, e1_ref,
             b1e_ref, n1b_ref, m_ref, k2_ref, c2v_ref, row2_ref,
             z_ref, b0_ref, b1_ref):
        a0b = a0_ref[...]
        a1b = a1_ref[...]
        sv = s0_ref[...] + s1_ref[...]
        deg = jnp.sum(a1b[:, UH - G:], axis=1, keepdims=True)
        h1 = (jnp.dot(a0b, w1aa_ref[...], preferred_element_type=jnp.float32)
              + jnp.dot(a1b, wcomb_ref[...], preferred_element_type=jnp.float32)
              + jnp.dot(sv, e1_ref[...], preferred_element_type=jnp.float32)
              + deg * b1e_ref[...]
              + n1b_ref[...])
        z_ref[...] = jnp.dot(h1, m_ref[...], preferred_element_type=jnp.float32)
        base = (jnp.dot(sv, k2_ref[...], preferred_element_type=jnp.float32)
                + deg * c2v_ref[...] + row2_ref[...])
        b0_ref[...] = base[:, :ZH]
        b1_ref[...] = base[:, ZH:]

    full = lambda shape: pl.BlockSpec(shape, lambda i: (0, 0))
    z, b0, b1 = pl.pallas_call(
        body,
        grid=(nblocks,),
        in_specs=[
            pl.BlockSpec((BN, UH), lambda i: (i, 0)),
            pl.BlockSpec((BN, UH), lambda i: (i, 0)),
            pl.BlockSpec((BN, EDGE_DIM), lambda i: (i, 0)),
            pl.BlockSpec((BN, EDGE_DIM), lambda i: (i, 0)),
            full((UH, HID)),
            full((UH, HID)),
            full((EDGE_DIM, HID)),
            full((1, HID)),
            full((1, HID)),
            full((HID, NODE_DIM)),
            full((EDGE_DIM, NODE_DIM)),
            full((1, NODE_DIM)),
            full((1, NODE_DIM)),
        ],
        out_specs=(
            pl.BlockSpec((BN, NODE_DIM), lambda i: (i, 0)),
            pl.BlockSpec((BN, ZH), lambda i: (i, 0)),
            pl.BlockSpec((BN, ZH), lambda i: (i, 0)),
        ),
        out_shape=(
            jax.ShapeDtypeStruct((N, NODE_DIM), jnp.float32),
            jax.ShapeDtypeStruct((N, ZH), jnp.float32),
            jax.ShapeDtypeStruct((N, ZH), jnp.float32),
        ),
    )(a0, a1, s0, s1, w1aa, wcomb, e1, b1e, n1_b2, m, k2, c2v, row2)
    return z, b0, b1


def kernel(x, edge_index, edge_attr, t, batch,
           e1_W, e1_b, n1_W, n1_b,
           e2_W, e2_b, n2_W, n2_b,
           out_W, out_b):
    f32 = jnp.float32
    src = edge_index[0].astype(jnp.int32)
    dst = edge_index[1].astype(jnp.int32)
    batch = batch.astype(jnp.int32)

    onehot = (batch[:, None] == jnp.arange(G, dtype=jnp.int32)[None, :]).astype(f32)
    u2 = jnp.concatenate([x, onehot], axis=1).reshape(2 * N, UH)
    pad_e = EP - E
    src_p = jnp.concatenate([src, jnp.zeros((pad_e,), jnp.int32)])
    trash = N + (jnp.arange(pad_e, dtype=jnp.int32) % 16)
    dst_p = jnp.concatenate([dst, trash])

    st = src_p.reshape(NS, NGRP, GB, CH)
    dt = dst_p.reshape(NS, NGRP, GB, CH)
    per_c = [jnp.stack([2 * st + c, dt], axis=3) for c in range(NC)]
    idx1 = jnp.concatenate(per_c).reshape(NC * NS * NGRP, 2 * GB, CH)
    dst2d = dst_p.reshape(NC * NS * SCH, CH)

    za = jnp.zeros((ISTRIPE, UH), f32)
    zs = jnp.zeros((ISTRIPE, EDGE_DIM), f32)

    a0, a1 = _sc_pass1(u2, idx1, za)
    s0, s1 = _sc_spass(edge_attr, dst2d, zs)

    t2, e1p, b1e, m, k2, c2v, row2 = _prep_weights(
        t.reshape(G, 1), n1_W, e1_W, e1_b.reshape(1, -1), n2_W,
        n2_b.reshape(1, -1), out_W, out_b.reshape(1, -1), e2_W,
        e2_b.reshape(1, -1))

    w1aa = n1_W[:UH, :]
    wcomb = jnp.concatenate([n1_W[UH:NODE_DIM, :], t2], axis=0)

    z, b0, b1 = _dense_mid(a0, a1, s0, s1, w1aa, wcomb, e1p, b1e,
                           n1_b.reshape(1, -1), m, k2, c2v, row2)

    z2 = z.reshape(2 * N, ZH)
    return _sc_pass2(z2, idx1, b0, b1)
